# Initial kernel scaffold; baseline (speedup 1.0000x reference)
import math

import jax
import jax.numpy as jnp
from jax import lax
from jax.experimental import pallas as pl
from jax.experimental.pallas import tpu as pltpu
from jax.experimental.pallas import tpu_sc as plsc

L = 65536
S = 128
NVOX = 128 ** 3
HALF = NVOX // 2
NC, NS = 2, 16
NW = NC * NS
NL1 = L // NW
NL2 = L // NS
G = 8

SIGMA = 300.0 * 0.15 / 2.355
INV_SIGMA = 1.0 / SIGMA
KW = 15.0 / (SIGMA * math.sqrt(2.0 * math.pi))

_mesh = plsc.VectorSubcoreMesh(core_axis_name="c", subcore_axis_name="s")


def _stage_and_precompute(base, n, p1x, p1y, p1z, p2x, p2y, p2z, tof,
                          bx, by, bz, bdx, bdy, bdz, blen, bk, bb):
    pltpu.sync_copy(p1x.at[pl.ds(base, n)], bx)
    pltpu.sync_copy(p1y.at[pl.ds(base, n)], by)
    pltpu.sync_copy(p1z.at[pl.ds(base, n)], bz)
    pltpu.sync_copy(p2x.at[pl.ds(base, n)], bdx)
    pltpu.sync_copy(p2y.at[pl.ds(base, n)], bdy)
    pltpu.sync_copy(p2z.at[pl.ds(base, n)], bdz)
    pltpu.sync_copy(tof.at[pl.ds(base, n)], bb)

    def pre(i, _):
        sl = pl.ds(i * 16, 16)
        dx = bdx[sl] - bx[sl]
        dy = bdy[sl] - by[sl]
        dz = bdz[sl] - bz[sl]
        n2 = dx * dx + dy * dy + dz * dz
        yh = plsc.bitcast(
            jnp.int32(0x5F3759DF) - (plsc.bitcast(n2, jnp.int32) >> 1),
            jnp.float32)
        for _ in range(3):
            yh = yh * (1.5 - 0.5 * n2 * yh * yh)
        ln = n2 * yh
        bdx[sl] = dx
        bdy[sl] = dy
        bdz[sl] = dz
        blen[sl] = ln
        bk[sl] = ln * (KW / S)
        bb[sl] = 0.5 * ln + bb[sl] * 0.15
        return 0

    lax.fori_loop(0, n // 16, pre, 0)


def _trace_samples(l, bx, by, bz, bdx, bdy, bdz):
    p1xs = bx[l]
    p1ys = by[l]
    p1zs = bz[l]
    dxs = bdx[l]
    dys = bdy[l]
    dzs = bdz[l]
    out = []
    for j in range(S // 16):
        t = (lax.iota(jnp.float32, 16) + (j * 16 + 0.5)) * (1.0 / S)
        px = p1xs + t * dxs
        py = p1ys + t * dys
        pz = p1zs + t * dzs
        ix = ((px + 128.0) * 0.5).astype(jnp.int32)
        iy = ((py + 128.0) * 0.5).astype(jnp.int32)
        iz = ((pz + 128.0) * 0.5).astype(jnp.int32)
        flat = (ix * 128 + iy) * 128 + iz
        out.append((j, t, flat))
    return out


def _proj_body(image, p1x, p1y, p1z, p2x, p2y, p2z, tof, proj_out,
               bx, by, bz, bdx, bdy, bdz, blen, bk, bb, bproj,
               idx_buf, w_buf, val_buf):
    c = lax.axis_index("c")
    s = lax.axis_index("s")
    wid = s * NC + c
    base = wid * NL1

    _stage_and_precompute(base, NL1, p1x, p1y, p1z, p2x, p2y, p2z, tof,
                          bx, by, bz, bdx, bdy, bdz, blen, bk, bb)

    def grp(g0, _):
        for gg in range(G):
            l = g0 * G + gg
            lns = blen[l]
            ks = bk[l]
            bs = bb[l]
            for j, t, flat in _trace_samples(l, bx, by, bz, bdx, bdy, bdz):
                e = (t * lns - bs) * INV_SIGMA
                w = jnp.exp(-0.5 * (e * e)) * ks
                jsl = pl.ds(j * 16, 16)
                idx_buf[gg, jsl] = flat
                w_buf[gg, jsl] = w
        pltpu.sync_copy(image.at[idx_buf], val_buf)
        for gg in range(G):
            acc = val_buf[gg, pl.ds(0, 16)] * w_buf[gg, pl.ds(0, 16)]
            for j in range(1, S // 16):
                jsl = pl.ds(j * 16, 16)
                acc = acc + val_buf[gg, jsl] * w_buf[gg, jsl]
            bproj[g0 * G + gg] = jnp.sum(acc)
        return 0

    lax.fori_loop(0, NL1 // G, grp, 0)
    pltpu.sync_copy(bproj, proj_out.at[pl.ds(base, NL1)])


def _bp_body(image, eff, p1x, p1y, p1z, p2x, p2y, p2z, tof, proj, out,
             bx, by, bz, bdx, bdy, bdz, blen, bk, bb, bproj,
             idx_buf, val_buf, acc):
    c = lax.axis_index("c")
    s = lax.axis_index("s")
    base = s * NL2
    half_base = c * HALF

    def z16(i, _):
        bx[pl.ds(i * 16, 16)] = jnp.zeros((16,), jnp.float32)
        return 0

    lax.fori_loop(0, NL2 // 16, z16, 0)
    nrep = HALF // NS // NL2
    for r in range(nrep):
        pltpu.sync_copy(bx, acc.at[pl.ds((s * nrep + r) * NL2, NL2)])
    plsc.subcore_barrier()

    _stage_and_precompute(base, NL2, p1x, p1y, p1z, p2x, p2y, p2z, tof,
                          bx, by, bz, bdx, bdy, bdz, blen, bk, bb)
    pltpu.sync_copy(proj.at[pl.ds(base, NL2)], bproj)

    def grp(g0, _):
        for gg in range(G):
            l = g0 * G + gg
            lns = blen[l]
            ks = bk[l] * bproj[l]
            bs = bb[l]
            for j, t, flat in _trace_samples(l, bx, by, bz, bdx, bdy, bdz):
                e = (t * lns - bs) * INV_SIGMA
                v = jnp.exp(-0.5 * (e * e)) * ks
                lf = flat - half_base
                m = (lf >= 0) & (lf < HALF)
                jsl = pl.ds(j * 16, 16)
                idx_buf[gg, jsl] = jnp.where(m, lf, 0)
                val_buf[gg, jsl] = jnp.where(m, v, 0.0)
        pltpu.sync_copy(val_buf, acc.at[idx_buf], add=True)
        return 0

    lax.fori_loop(0, NL2 // G, grp, 0)
    plsc.subcore_barrier()

    for r in range(nrep):
        off = (s * nrep + r) * NL2
        goff = half_base + off
        pltpu.sync_copy(acc.at[pl.ds(off, NL2)], bx)
        pltpu.sync_copy(image.at[pl.ds(goff, NL2)], by)
        pltpu.sync_copy(eff.at[pl.ds(goff, NL2)], bz)

        def fin(i, _):
            sl = pl.ds(i * 16, 16)
            bdx[sl] = by[sl] / (bz[sl] + 1e-8) * bx[sl]
            return 0

        lax.fori_loop(0, NL2 // 16, fin, 0)
        pltpu.sync_copy(bdx, out.at[pl.ds(goff, NL2)])


_proj_call = pl.kernel(
    _proj_body,
    out_type=jax.ShapeDtypeStruct((L,), jnp.float32),
    mesh=_mesh,
    scratch_types=[pltpu.VMEM((NL1,), jnp.float32)] * 10 + [
        pltpu.VMEM((G, S), jnp.int32),
        pltpu.VMEM((G, S), jnp.float32),
        pltpu.VMEM((G, S), jnp.float32),
    ],
)

_bp_call = pl.kernel(
    _bp_body,
    out_type=jax.ShapeDtypeStruct((NVOX,), jnp.float32),
    mesh=_mesh,
    scratch_types=[pltpu.VMEM((NL2,), jnp.float32)] * 10 + [
        pltpu.VMEM((G, S), jnp.int32),
        pltpu.VMEM((G, S), jnp.float32),
        pltpu.VMEM_SHARED((HALF,), jnp.float32),
    ],
)


def kernel(image, efficiency_map, lors):
    image_flat = image.reshape(-1)
    eff_flat = efficiency_map.reshape(-1)
    p1x, p1y, p1z, p2x, p2y, p2z, tof = [lors[:, i] for i in range(7)]
    proj = _proj_call(image_flat, p1x, p1y, p1z, p2x, p2y, p2z, tof)
    res = _bp_call(image_flat, eff_flat, p1x, p1y, p1z, p2x, p2y, p2z, tof,
                   proj)
    return res.reshape(image.shape)


# speedup vs baseline: 30.6362x; 30.6362x over previous
import math

import jax
import jax.numpy as jnp
from jax import lax
from jax.experimental import pallas as pl
from jax.experimental.pallas import tpu as pltpu
from jax.experimental.pallas import tpu_sc as plsc

L = 65536
S = 128
NVOX = 128 ** 3
HALF = NVOX // 2
NC, NS = 2, 16
NW = NC * NS
NL1 = L // NW
NL2 = L // NS
G = 16

SIGMA = 300.0 * 0.15 / 2.355
INV_SIGMA = 1.0 / SIGMA
KW = 15.0 / (SIGMA * math.sqrt(2.0 * math.pi))

_mesh = plsc.VectorSubcoreMesh(core_axis_name="c", subcore_axis_name="s")
_params = pltpu.CompilerParams(needs_layout_passes=False)


def _stage_and_precompute(base, n, p1x, p1y, p1z, p2x, p2y, p2z, tof,
                          bx, by, bz, bdx, bdy, bdz, blen, bk, bb):
    pltpu.sync_copy(p1x.at[pl.ds(base, n)], bx)
    pltpu.sync_copy(p1y.at[pl.ds(base, n)], by)
    pltpu.sync_copy(p1z.at[pl.ds(base, n)], bz)
    pltpu.sync_copy(p2x.at[pl.ds(base, n)], bdx)
    pltpu.sync_copy(p2y.at[pl.ds(base, n)], bdy)
    pltpu.sync_copy(p2z.at[pl.ds(base, n)], bdz)
    pltpu.sync_copy(tof.at[pl.ds(base, n)], bb)

    def pre(i, _):
        sl = pl.ds(i * 16, 16)
        dx = bdx[sl] - bx[sl]
        dy = bdy[sl] - by[sl]
        dz = bdz[sl] - bz[sl]
        n2 = dx * dx + dy * dy + dz * dz
        yh = plsc.bitcast(
            jnp.int32(0x5F3759DF) - (plsc.bitcast(n2, jnp.int32) >> 1),
            jnp.float32)
        for _ in range(3):
            yh = yh * (1.5 - 0.5 * n2 * yh * yh)
        ln = n2 * yh
        bdx[sl] = dx
        bdy[sl] = dy
        bdz[sl] = dz
        blen[sl] = ln
        bk[sl] = ln * (KW / S)
        bb[sl] = 0.5 * ln + bb[sl] * 0.15
        return 0

    lax.fori_loop(0, n // 16, pre, 0)


def _load_group(g0, bx, by, bz, bdx, bdy, bdz, blen, bk, bb):
    sl = pl.ds(g0 * 16, 16)
    return (bx[sl], by[sl], bz[sl], bdx[sl], bdy[sl], bdz[sl],
            blen[sl], bk[sl], bb[sl])


def _trace_samples(gg, vx, vy, vz, vdx, vdy, vdz):
    p1xs = vx[gg]
    p1ys = vy[gg]
    p1zs = vz[gg]
    dxs = vdx[gg]
    dys = vdy[gg]
    dzs = vdz[gg]
    out = []
    for j in range(S // 16):
        t = (lax.iota(jnp.int32, 16).astype(jnp.float32)
             + (j * 16 + 0.5)) * (1.0 / S)
        px = p1xs + t * dxs
        py = p1ys + t * dys
        pz = p1zs + t * dzs
        ix = ((px + 128.0) * 0.5).astype(jnp.int32)
        iy = ((py + 128.0) * 0.5).astype(jnp.int32)
        iz = ((pz + 128.0) * 0.5).astype(jnp.int32)
        flat = (ix * 128 + iy) * 128 + iz
        out.append((j, t, flat))
    return out


def _proj_body(image, p1x, p1y, p1z, p2x, p2y, p2z, tof, proj_out,
               bx, by, bz, bdx, bdy, bdz, blen, bk, bb, bproj,
               idx_buf, w_buf, val_buf, sem):
    c = lax.axis_index("c")
    s = lax.axis_index("s")
    wid = s * NC + c
    base = wid * NL1

    _stage_and_precompute(base, NL1, p1x, p1y, p1z, p2x, p2y, p2z, tof,
                          bx, by, bz, bdx, bdy, bdz, blen, bk, bb)

    lane = lax.iota(jnp.int32, 16)

    def grp(g0, _):
        (vx, vy, vz, vdx, vdy, vdz, vlen, vk, vb) = _load_group(
            g0, bx, by, bz, bdx, bdy, bdz, blen, bk, bb)
        for gg in range(G):
            lns = vlen[gg]
            ks = vk[gg]
            bs = vb[gg]
            for j, t, flat in _trace_samples(gg, vx, vy, vz, vdx, vdy, vdz):
                e = (t * lns - bs) * INV_SIGMA
                w = jnp.exp(-0.5 * (e * e)) * ks
                jsl = pl.ds(j * 16, 16)
                idx_buf[gg, jsl] = flat
                w_buf[gg, jsl] = w
        descs = [pltpu.async_copy(image.at[idx_buf.at[gg]], val_buf.at[gg],
                                  sem) for gg in range(G)]
        for d in descs:
            d.wait()
        pvec = jnp.zeros((16,), jnp.float32)
        for gg in range(G):
            acc = val_buf[gg, pl.ds(0, 16)] * w_buf[gg, pl.ds(0, 16)]
            for j in range(1, S // 16):
                jsl = pl.ds(j * 16, 16)
                acc = acc + val_buf[gg, jsl] * w_buf[gg, jsl]
            pvec = jnp.where(lane == gg, jnp.sum(acc), pvec)
        bproj[pl.ds(g0 * 16, 16)] = pvec
        return 0

    lax.fori_loop(0, NL1 // G, grp, 0)
    pltpu.sync_copy(bproj, proj_out.at[pl.ds(base, NL1)])


def _bp_body(image, eff, p1x, p1y, p1z, p2x, p2y, p2z, tof, proj, out,
             bx, by, bz, bdx, bdy, bdz, blen, bk, bb, bproj,
             idx_buf, val_buf, acc, sem):
    c = lax.axis_index("c")
    s = lax.axis_index("s")
    base = s * NL2
    half_base = c * HALF

    def z16(i, _):
        bx[pl.ds(i * 16, 16)] = jnp.zeros((16,), jnp.float32)
        return 0

    lax.fori_loop(0, NL2 // 16, z16, 0)
    nrep = HALF // NS // NL2
    for r in range(nrep):
        pltpu.sync_copy(bx, acc.at[pl.ds((s * nrep + r) * NL2, NL2)])
    plsc.subcore_barrier()

    _stage_and_precompute(base, NL2, p1x, p1y, p1z, p2x, p2y, p2z, tof,
                          bx, by, bz, bdx, bdy, bdz, blen, bk, bb)
    pltpu.sync_copy(proj.at[pl.ds(base, NL2)], bproj)

    def grp(g0, _):
        (vx, vy, vz, vdx, vdy, vdz, vlen, vk, vb) = _load_group(
            g0, bx, by, bz, bdx, bdy, bdz, blen, bk, bb)
        vk2 = vk * bproj[pl.ds(g0 * 16, 16)]
        for gg in range(G):
            lns = vlen[gg]
            ks = vk2[gg]
            bs = vb[gg]
            for j, t, flat in _trace_samples(gg, vx, vy, vz, vdx, vdy, vdz):
                e = (t * lns - bs) * INV_SIGMA
                v = jnp.exp(-0.5 * (e * e)) * ks
                lf = flat - half_base
                m = (lf >= 0) & (lf < HALF)
                jsl = pl.ds(j * 16, 16)
                idx_buf[gg, jsl] = jnp.where(m, lf, 0)
                val_buf[gg, jsl] = jnp.where(m, v, 0.0)
        descs = [pltpu.async_copy(val_buf.at[gg], acc.at[idx_buf.at[gg]],
                                  sem, add=True) for gg in range(G)]
        for d in descs:
            d.wait()
        return 0

    lax.fori_loop(0, NL2 // G, grp, 0)
    plsc.subcore_barrier()

    for r in range(nrep):
        off = (s * nrep + r) * NL2
        goff = half_base + off
        pltpu.sync_copy(acc.at[pl.ds(off, NL2)], bx)
        pltpu.sync_copy(image.at[pl.ds(goff, NL2)], by)
        pltpu.sync_copy(eff.at[pl.ds(goff, NL2)], bz)

        def fin(i, _):
            sl = pl.ds(i * 16, 16)
            bdx[sl] = by[sl] / (bz[sl] + 1e-8) * bx[sl]
            return 0

        lax.fori_loop(0, NL2 // 16, fin, 0)
        pltpu.sync_copy(bdx, out.at[pl.ds(goff, NL2)])


_proj_call = pl.kernel(
    _proj_body,
    out_type=jax.ShapeDtypeStruct((L,), jnp.float32),
    mesh=_mesh,
    compiler_params=_params,
    scratch_types=[pltpu.VMEM((NL1,), jnp.float32)] * 10 + [
        pltpu.VMEM((G, S), jnp.int32),
        pltpu.VMEM((G, S), jnp.float32),
        pltpu.VMEM((G, S), jnp.float32),
        pltpu.SemaphoreType.DMA,
    ],
)

_bp_call = pl.kernel(
    _bp_body,
    out_type=jax.ShapeDtypeStruct((NVOX,), jnp.float32),
    mesh=_mesh,
    compiler_params=_params,
    scratch_types=[pltpu.VMEM((NL2,), jnp.float32)] * 10 + [
        pltpu.VMEM((G, S), jnp.int32),
        pltpu.VMEM((G, S), jnp.float32),
        pltpu.VMEM_SHARED((HALF,), jnp.float32),
        pltpu.SemaphoreType.DMA,
    ],
)


def kernel(image, efficiency_map, lors):
    image_flat = image.reshape(-1)
    eff_flat = efficiency_map.reshape(-1)
    p1x, p1y, p1z, p2x, p2y, p2z, tof = [lors[:, i] for i in range(7)]
    proj = _proj_call(image_flat, p1x, p1y, p1z, p2x, p2y, p2z, tof)
    res = _bp_call(image_flat, eff_flat, p1x, p1y, p1z, p2x, p2y, p2z, tof,
                   proj)
    return res.reshape(image.shape)
